# baseline (speedup 1.0000x reference)
import jax
import jax.numpy as jnp
from jax import lax
from jax.experimental import pallas as pl
from jax.experimental.pallas import tpu as pltpu
from jax.experimental.pallas import tpu_sc as plsc

BATCH = 16384
LATENT_DIM = 32
NUM_CORES = 2
NUM_SUBCORES = 16
NUM_WORKERS = NUM_CORES * NUM_SUBCORES
B_PER_W = BATCH // NUM_WORKERS
LANES = 16
GROUPS = B_PER_W // LANES


def _gmf_body(uidx_hbm, iidx_hbm, utab_hbm, itab_hbm, fcw_hbm, fcb_hbm,
              out_hbm, uidx_v, iidx_v, urows_v, irows_v, w_v, b_v, out_v,
              sem):
    wid = lax.axis_index("s") * NUM_CORES + lax.axis_index("c")
    base = wid * B_PER_W

    pltpu.sync_copy(uidx_hbm.at[pl.ds(base, B_PER_W)], uidx_v)
    pltpu.sync_copy(iidx_hbm.at[pl.ds(base, B_PER_W)], iidx_v)
    cp_u = pltpu.async_copy(utab_hbm.at[uidx_v], urows_v, sem)
    cp_i = pltpu.async_copy(itab_hbm.at[iidx_v], irows_v, sem)
    pltpu.sync_copy(fcw_hbm, w_v)
    pltpu.sync_copy(fcb_hbm, b_v)
    cp_u.wait()
    cp_i.wait()

    w_lo = w_v[pl.ds(0, LANES)]
    w_hi = w_v[pl.ds(LANES, LANES)]
    bias = b_v[pl.ds(0, LANES)][0]
    lane_iota = lax.iota(jnp.int32, LANES)

    def group(g, carry):
        rows = g * LANES + lane_iota
        acc = jnp.zeros((LANES,), jnp.float32)
        for d in range(LATENT_DIM):
            col = jnp.full((LANES,), d, jnp.int32)
            gu = plsc.load_gather(urows_v, [rows, col])
            gi = plsc.load_gather(irows_v, [rows, col])
            w_d = w_lo[d] if d < LANES else w_hi[d - LANES]
            acc = acc + gu * gi * w_d
        out_v[pl.ds(g * LANES, LANES)] = acc + bias
        return carry

    lax.fori_loop(0, GROUPS, group, 0)

    pltpu.sync_copy(out_v, out_hbm.at[pl.ds(base, B_PER_W)])


def kernel(user_indices, item_indices, user_table, item_table, fc_w, fc_b):
    mesh = plsc.VectorSubcoreMesh(core_axis_name="c", subcore_axis_name="s")
    run = pl.kernel(
        _gmf_body,
        out_type=jax.ShapeDtypeStruct((BATCH,), jnp.float32),
        mesh=mesh,
        compiler_params=pltpu.CompilerParams(
            needs_layout_passes=False, use_tc_tiling_on_sc=False),
        scratch_types=[
            pltpu.VMEM((B_PER_W,), jnp.int32),
            pltpu.VMEM((B_PER_W,), jnp.int32),
            pltpu.VMEM((B_PER_W, LATENT_DIM), jnp.float32),
            pltpu.VMEM((B_PER_W, LATENT_DIM), jnp.float32),
            pltpu.VMEM((LATENT_DIM,), jnp.float32),
            pltpu.VMEM((LANES,), jnp.float32),
            pltpu.VMEM((B_PER_W,), jnp.float32),
            pltpu.SemaphoreType.DMA,
        ],
    )
    out = run(user_indices, item_indices, user_table, item_table,
              jnp.reshape(fc_w, (LATENT_DIM,)),
              jnp.broadcast_to(jnp.reshape(fc_b, (1,)), (LANES,)))
    return jnp.reshape(out, (BATCH, 1))


# speedup vs baseline: 1.5551x; 1.5551x over previous
import jax
import jax.numpy as jnp
from jax import lax
from jax.experimental import pallas as pl
from jax.experimental.pallas import tpu as pltpu
from jax.experimental.pallas import tpu_sc as plsc

BATCH = 16384
D = 32
NUM_ROWS = 1000000
NC = 2
NS = 16
NW = NC * NS
LANES = 16
COLS_PER_W = 31232
STEP = 512
FULL_STEPS = COLS_PER_W // STEP
LAST_LO = 31 * COLS_PER_W
LAST_FULL_STEPS = (NUM_ROWS - LAST_LO) // STEP
TAIL_LO = LAST_LO + LAST_FULL_STEPS * STEP
TAIL_COLS = NUM_ROWS - TAIL_LO
LIST_CAP = BATCH + LANES
B_PER_W = BATCH // NW


def _select_body(uidx_hbm, iidx_hbm, utab_hbm, itab_hbm,
                 uflat_hbm, iflat_hbm,
                 uidx_v, iidx_v, ulist_v, ilist_v, ubuf, ibuf, tail_v,
                 stage_v, sems, dsem):
    wid = lax.axis_index("s") * NC + lax.axis_index("c")
    lo = wid * COLS_PER_W
    hi = jnp.where(wid == NW - 1, NUM_ROWS, lo + COLS_PER_W)

    pltpu.sync_copy(uidx_hbm, uidx_v)
    pltpu.sync_copy(iidx_hbm, iidx_v)

    lane_iota = lax.iota(jnp.int32, LANES)

    def scan_indices(idx_v, list_v):
        def body(g, n):
            v = idx_v[pl.ds(g * LANES, LANES)]
            m = (v >= lo) & (v < hi)
            entry = ((v - lo) << 14) | (g * LANES + lane_iota)
            plsc.store_compressed(list_v.at[pl.ds(n, LANES)], entry, mask=m)
            cnt = plsc.all_reduce_population_count(m)
            return n + cnt[0]

        n = lax.fori_loop(0, BATCH // LANES, body, jnp.int32(0))
        list_v[pl.ds(n, LANES)] = jnp.zeros((LANES,), jnp.int32)
        return n

    nu = scan_indices(uidx_v, ulist_v)
    ni = scan_indices(iidx_v, ilist_v)

    def scan_matches(list_v, n, buf, ncols, s_lo, out_hbm, state):
        ngroups = (n + LANES - 1) // LANES

        def group(g, st):
            e = list_v[pl.ds(g * LANES, LANES)]
            valid = (g * LANES + lane_iota) < n
            u_loc = lax.shift_right_logical(e, 14)
            b = e & 0x3FFF
            m = valid & (u_loc >= s_lo) & (u_loc < s_lo + ncols)

            m_i = m.astype(jnp.int32)

            def emit(st2):
                st3 = st2
                for j in range(LANES):
                    st3 = emit_lane(j, m_i, u_loc, b, buf, s_lo, out_hbm,
                                    st3)
                return st3

            return cond_carry(jnp.any(m), emit, st)

        return lax.fori_loop(0, ngroups, group, state)

    def cond_carry(pred, fn, st):
        return lax.cond(pred, fn, lambda s: s, st)

    def emit_lane(j, m, u_loc, b, buf, s_lo, out_hbm, st):
        def fire(st2):
            def drain(st3):
                pltpu.make_async_copy(
                    stage_v.at[j], out_hbm.at[pl.ds(0, D)], sems.at[j]
                ).wait()
                return st3

            st2 = cond_carry(((st2 >> j) & 1) == 1, drain, st2)
            col = jnp.full((LANES,), u_loc[j] - s_lo, jnp.int32)
            v_lo = plsc.load_gather(buf, [lane_iota, col])
            v_hi = plsc.load_gather(buf, [lane_iota + LANES, col])
            stage_v[j, pl.ds(0, LANES)] = v_lo
            stage_v[j, pl.ds(LANES, LANES)] = v_hi
            pltpu.async_copy(
                stage_v.at[j], out_hbm.at[pl.ds(b[j] * D, D)], sems.at[j])
            return st2 | (1 << j)

        return cond_carry(m[j] == 1, fire, st)

    def step_body(s, st):
        def run(st2):
            c0 = lo + s * STEP
            pltpu.sync_copy(utab_hbm.at[:, pl.ds(c0, STEP)], ubuf)
            st2 = scan_matches(ulist_v, nu, ubuf, STEP, s * STEP,
                               uflat_hbm, st2)
            pltpu.sync_copy(itab_hbm.at[:, pl.ds(c0, STEP)], ibuf)
            st2 = scan_matches(ilist_v, ni, ibuf, STEP, s * STEP,
                               iflat_hbm, st2)
            return st2

        nsteps = jnp.where(wid == NW - 1, LAST_FULL_STEPS, FULL_STEPS)
        return cond_carry(s < nsteps, run, st)

    state = lax.fori_loop(0, LAST_FULL_STEPS, step_body, jnp.int32(0))

    def tail(st):
        s_lo = TAIL_LO - LAST_LO
        pltpu.sync_copy(utab_hbm.at[:, pl.ds(TAIL_LO, TAIL_COLS)], tail_v)
        st = scan_matches(ulist_v, nu, tail_v, TAIL_COLS, s_lo,
                          uflat_hbm, st)
        pltpu.sync_copy(itab_hbm.at[:, pl.ds(TAIL_LO, TAIL_COLS)], tail_v)
        st = scan_matches(ilist_v, ni, tail_v, TAIL_COLS, s_lo,
                          iflat_hbm, st)
        return st

    state = cond_carry(wid == NW - 1, tail, state)

    def final_drain(st):
        for j in range(LANES):
            def drain(s3, j=j):
                pltpu.make_async_copy(
                    stage_v.at[j], uflat_hbm.at[pl.ds(0, D)], sems.at[j]
                ).wait()
                return s3

            st = cond_carry(((st >> j) & 1) == 1, drain, st)
        return st

    final_drain(state)


def _combine_body(uidx_hbm, iidx_hbm, uflat_hbm, iflat_hbm, fcw_hbm,
                  fcb_hbm, out_hbm, uvals_v, ivals_v, w_v, b_v, out_v, sem):
    wid = lax.axis_index("s") * NC + lax.axis_index("c")
    base = wid * B_PER_W

    cu = pltpu.async_copy(
        uflat_hbm.at[pl.ds(base * D, B_PER_W * D)], uvals_v, sem)
    ci = pltpu.async_copy(
        iflat_hbm.at[pl.ds(base * D, B_PER_W * D)], ivals_v, sem)
    pltpu.sync_copy(fcw_hbm, w_v)
    pltpu.sync_copy(fcb_hbm, b_v)
    cu.wait()
    ci.wait()

    w_lo = w_v[pl.ds(0, LANES)]
    w_hi = w_v[pl.ds(LANES, LANES)]
    bias = b_v[pl.ds(0, LANES)][0]
    row_iota = lax.iota(jnp.int32, LANES) * D

    def group(g, carry):
        acc = jnp.zeros((LANES,), jnp.float32)
        base_flat = g * (LANES * D)
        for d in range(D):
            idxv = row_iota + (base_flat + d)
            gu = plsc.load_gather(uvals_v, [idxv])
            gi = plsc.load_gather(ivals_v, [idxv])
            w_d = w_lo[d] if d < LANES else w_hi[d - LANES]
            acc = acc + gu * gi * w_d
        out_v[pl.ds(g * LANES, LANES)] = acc + bias
        return carry

    lax.fori_loop(0, B_PER_W // LANES, group, 0)

    pltpu.sync_copy(out_v, out_hbm.at[pl.ds(base, B_PER_W)])


def kernel(user_indices, item_indices, user_table, item_table, fc_w, fc_b):
    mesh = plsc.VectorSubcoreMesh(core_axis_name="c", subcore_axis_name="s")
    params = pltpu.CompilerParams(needs_layout_passes=False)

    select = pl.kernel(
        _select_body,
        out_type=(
            jax.ShapeDtypeStruct((BATCH * D,), jnp.float32),
            jax.ShapeDtypeStruct((BATCH * D,), jnp.float32),
        ),
        mesh=mesh,
        compiler_params=params,
        scratch_types=[
            pltpu.VMEM((BATCH,), jnp.int32),
            pltpu.VMEM((BATCH,), jnp.int32),
            pltpu.VMEM((LIST_CAP,), jnp.int32),
            pltpu.VMEM((LIST_CAP,), jnp.int32),
            pltpu.VMEM((D, STEP), jnp.float32),
            pltpu.VMEM((D, STEP), jnp.float32),
            pltpu.VMEM((D, TAIL_COLS), jnp.float32),
            pltpu.VMEM((LANES, D), jnp.float32),
            pltpu.SemaphoreType.DMA((LANES,)),
            pltpu.SemaphoreType.DMA,
        ],
    )
    uflat, iflat = select(user_indices, item_indices,
                          user_table.T, item_table.T)

    combine = pl.kernel(
        _combine_body,
        out_type=jax.ShapeDtypeStruct((BATCH,), jnp.float32),
        mesh=mesh,
        compiler_params=params,
        scratch_types=[
            pltpu.VMEM((B_PER_W * D,), jnp.float32),
            pltpu.VMEM((B_PER_W * D,), jnp.float32),
            pltpu.VMEM((D,), jnp.float32),
            pltpu.VMEM((LANES,), jnp.float32),
            pltpu.VMEM((B_PER_W,), jnp.float32),
            pltpu.SemaphoreType.DMA,
        ],
    )
    out = combine(user_indices, item_indices, uflat, iflat,
                  jnp.reshape(fc_w, (D,)),
                  jnp.broadcast_to(jnp.reshape(fc_b, (1,)), (LANES,)))
    return jnp.reshape(out, (BATCH, 1))


# speedup vs baseline: 1.9856x; 1.2768x over previous
import jax
import jax.numpy as jnp
from jax import lax
from jax.experimental import pallas as pl
from jax.experimental.pallas import tpu as pltpu
from jax.experimental.pallas import tpu_sc as plsc

BATCH = 16384
D = 32
NUM_ROWS = 1000000
NC = 2
NS = 16
NW = NC * NS
LANES = 16
COLS_PER_W = 31232
STEP = 1024
FULL_STEPS = 30
LAST_LO = 31 * COLS_PER_W
TAIL_LO = 999936
TAIL_COLS = NUM_ROWS - TAIL_LO
LIST_CAP = BATCH + LANES
B_PER_W = BATCH // NW


def _select_body(uidx_hbm, iidx_hbm, utab_hbm, itab_hbm,
                 uflat_hbm, iflat_hbm,
                 idx_v, ulist_v, ilist_v, ubuf, ibuf, tail_v,
                 stage_v, sems, usem, isem):
    wid = lax.axis_index("s") * NC + lax.axis_index("c")
    last = wid == NW - 1
    lo = wid * COLS_PER_W
    hi = jnp.where(last, NUM_ROWS, lo + COLS_PER_W)

    lane_iota = lax.iota(jnp.int32, LANES)

    def scan_indices(idx_hbm, list_v):
        pltpu.sync_copy(idx_hbm, idx_v)

        def body(g, n):
            v = idx_v[pl.ds(g * LANES, LANES)]
            m = (v >= lo) & (v < hi)
            entry = ((v - lo) << 14) | (g * LANES + lane_iota)
            plsc.store_compressed(list_v.at[pl.ds(n, LANES)], entry, mask=m)
            cnt = plsc.all_reduce_population_count(m)
            return n + cnt[0]

        n = lax.fori_loop(0, BATCH // LANES, body, jnp.int32(0))
        list_v[pl.ds(n, LANES)] = jnp.zeros((LANES,), jnp.int32)
        return n

    nu = scan_indices(uidx_hbm, ulist_v)
    ni = scan_indices(iidx_hbm, ilist_v)

    def cond_carry(pred, fn, st):
        return lax.cond(pred, fn, lambda s: s, st)

    def emit_lane(j, m, u_loc, b, buf, s_lo, out_hbm, st):
        def fire(st2):
            def drain(st3):
                pltpu.make_async_copy(
                    stage_v.at[j], out_hbm.at[pl.ds(0, D)], sems.at[j]
                ).wait()
                return st3

            st2 = cond_carry(((st2 >> j) & 1) == 1, drain, st2)
            col = jnp.full((LANES,), u_loc[j] - s_lo, jnp.int32)
            v_lo = plsc.load_gather(buf, [lane_iota, col])
            v_hi = plsc.load_gather(buf, [lane_iota + LANES, col])
            stage_v[j, pl.ds(0, LANES)] = v_lo
            stage_v[j, pl.ds(LANES, LANES)] = v_hi
            pltpu.async_copy(
                stage_v.at[j], out_hbm.at[pl.ds(b[j] * D, D)], sems.at[j])
            return st2 | (1 << j)

        return cond_carry(m[j] == 1, fire, st)

    def scan_matches(list_v, n, buf, ncols, s_lo, out_hbm, state):
        ngroups = (n + LANES - 1) // LANES

        def group(g, st):
            e = list_v[pl.ds(g * LANES, LANES)]
            valid = (g * LANES + lane_iota) < n
            u_loc = lax.shift_right_logical(e, 14)
            b = e & 0x3FFF
            m = valid & (u_loc >= s_lo) & (u_loc < s_lo + ncols)
            m_i = m.astype(jnp.int32)

            def emit(st2):
                st3 = st2
                for j in range(LANES):
                    st3 = emit_lane(j, m_i, u_loc, b, buf, s_lo, out_hbm,
                                    st3)
                return st3

            return cond_carry(jnp.any(m), emit, st)

        return lax.fori_loop(0, ngroups, group, state)

    def do_step(c0, s_lo, ncols, ub_dst, ib_dst, st):
        cu = pltpu.async_copy(utab_hbm.at[:, c0], ub_dst, usem)
        ci = pltpu.async_copy(itab_hbm.at[:, c0], ib_dst, isem)
        cu.wait()
        st = scan_matches(ulist_v, nu, ubuf, ncols, s_lo, uflat_hbm, st)
        ci.wait()
        st = scan_matches(ilist_v, ni, ibuf, ncols, s_lo, iflat_hbm, st)
        return st

    def step_body(s, st):
        return do_step(pl.ds(lo + s * STEP, STEP), s * STEP, STEP,
                       ubuf, ibuf, st)

    state = lax.fori_loop(0, FULL_STEPS, step_body, jnp.int32(0))

    def normal_end(st):
        return do_step(pl.ds(lo + FULL_STEPS * STEP, 512),
                       FULL_STEPS * STEP, 512,
                       ubuf.at[:, pl.ds(0, 512)],
                       ibuf.at[:, pl.ds(0, 512)], st)

    def last_end(st):
        st = do_step(pl.ds(lo + FULL_STEPS * STEP, STEP),
                     FULL_STEPS * STEP, STEP, ubuf, ibuf, st)
        s_lo = TAIL_LO - LAST_LO
        pltpu.sync_copy(utab_hbm.at[:, pl.ds(TAIL_LO, TAIL_COLS)], tail_v)
        st = scan_matches(ulist_v, nu, tail_v, TAIL_COLS, s_lo,
                          uflat_hbm, st)
        pltpu.sync_copy(itab_hbm.at[:, pl.ds(TAIL_LO, TAIL_COLS)], tail_v)
        st = scan_matches(ilist_v, ni, tail_v, TAIL_COLS, s_lo,
                          iflat_hbm, st)
        return st

    state = lax.cond(last, last_end, normal_end, state)

    def final_drain(st):
        for j in range(LANES):
            def drain(s3, j=j):
                pltpu.make_async_copy(
                    stage_v.at[j], uflat_hbm.at[pl.ds(0, D)], sems.at[j]
                ).wait()
                return s3

            st = cond_carry(((st >> j) & 1) == 1, drain, st)
        return st

    final_drain(state)


def _combine_body(uidx_hbm, iidx_hbm, uflat_hbm, iflat_hbm, fcw_hbm,
                  fcb_hbm, out_hbm, uvals_v, ivals_v, w_v, b_v, out_v, sem):
    wid = lax.axis_index("s") * NC + lax.axis_index("c")
    base = wid * B_PER_W

    cu = pltpu.async_copy(
        uflat_hbm.at[pl.ds(base * D, B_PER_W * D)], uvals_v, sem)
    ci = pltpu.async_copy(
        iflat_hbm.at[pl.ds(base * D, B_PER_W * D)], ivals_v, sem)
    pltpu.sync_copy(fcw_hbm, w_v)
    pltpu.sync_copy(fcb_hbm, b_v)
    cu.wait()
    ci.wait()

    w_lo = w_v[pl.ds(0, LANES)]
    w_hi = w_v[pl.ds(LANES, LANES)]
    bias = b_v[pl.ds(0, LANES)][0]
    row_iota = lax.iota(jnp.int32, LANES) * D

    def group(g, carry):
        acc = jnp.zeros((LANES,), jnp.float32)
        base_flat = g * (LANES * D)
        for d in range(D):
            idxv = row_iota + (base_flat + d)
            gu = plsc.load_gather(uvals_v, [idxv])
            gi = plsc.load_gather(ivals_v, [idxv])
            w_d = w_lo[d] if d < LANES else w_hi[d - LANES]
            acc = acc + gu * gi * w_d
        out_v[pl.ds(g * LANES, LANES)] = acc + bias
        return carry

    lax.fori_loop(0, B_PER_W // LANES, group, 0)

    pltpu.sync_copy(out_v, out_hbm.at[pl.ds(base, B_PER_W)])


def kernel(user_indices, item_indices, user_table, item_table, fc_w, fc_b):
    mesh = plsc.VectorSubcoreMesh(core_axis_name="c", subcore_axis_name="s")
    params = pltpu.CompilerParams(needs_layout_passes=False)

    select = pl.kernel(
        _select_body,
        out_type=(
            jax.ShapeDtypeStruct((BATCH * D,), jnp.float32),
            jax.ShapeDtypeStruct((BATCH * D,), jnp.float32),
        ),
        mesh=mesh,
        compiler_params=params,
        scratch_types=[
            pltpu.VMEM((BATCH,), jnp.int32),
            pltpu.VMEM((LIST_CAP,), jnp.int32),
            pltpu.VMEM((LIST_CAP,), jnp.int32),
            pltpu.VMEM((D, STEP), jnp.float32),
            pltpu.VMEM((D, STEP), jnp.float32),
            pltpu.VMEM((D, TAIL_COLS), jnp.float32),
            pltpu.VMEM((LANES, D), jnp.float32),
            pltpu.SemaphoreType.DMA((LANES,)),
            pltpu.SemaphoreType.DMA,
            pltpu.SemaphoreType.DMA,
        ],
    )
    uflat, iflat = select(user_indices, item_indices,
                          user_table.T, item_table.T)

    combine = pl.kernel(
        _combine_body,
        out_type=jax.ShapeDtypeStruct((BATCH,), jnp.float32),
        mesh=mesh,
        compiler_params=params,
        scratch_types=[
            pltpu.VMEM((B_PER_W * D,), jnp.float32),
            pltpu.VMEM((B_PER_W * D,), jnp.float32),
            pltpu.VMEM((D,), jnp.float32),
            pltpu.VMEM((LANES,), jnp.float32),
            pltpu.VMEM((B_PER_W,), jnp.float32),
            pltpu.SemaphoreType.DMA,
        ],
    )
    out = combine(user_indices, item_indices, uflat, iflat,
                  jnp.reshape(fc_w, (D,)),
                  jnp.broadcast_to(jnp.reshape(fc_b, (1,)), (LANES,)))
    return jnp.reshape(out, (BATCH, 1))


# speedup vs baseline: 3.5756x; 1.8008x over previous
import jax
import jax.numpy as jnp
from jax import lax
from jax.experimental import pallas as pl
from jax.experimental.pallas import tpu as pltpu
from jax.experimental.pallas import tpu_sc as plsc

BATCH = 16384
D = 32
NUM_ROWS = 1000000
NC = 2
NS = 16
NW = NC * NS
LANES = 16
COLS_PER_W = 31232
STEP = 1024
FULL_STEPS = 30
LAST_LO = 31 * COLS_PER_W
TAIL_LO = 999936
TAIL_COLS = NUM_ROWS - TAIL_LO
NBUK = 32
LIST_CAP = BATCH + LANES
B_PER_W = BATCH // NW


def _select_body(uidx_hbm, iidx_hbm, utab_hbm, itab_hbm,
                 uflat_hbm, iflat_hbm,
                 idx_v, ulist_v, ilist_v, ubuf, ibuf, tail_v,
                 stage_v, ucnt_v, ubase_v, icnt_v, ibase_v, cur_v,
                 sems, usem, isem):
    wid = lax.axis_index("s") * NC + lax.axis_index("c")
    last = wid == NW - 1
    lo = wid * COLS_PER_W
    hi = jnp.where(last, NUM_ROWS, lo + COLS_PER_W)

    lane_iota = lax.iota(jnp.int32, LANES)
    ones_i = jnp.full((LANES,), 1, jnp.int32)
    zeros_i = jnp.zeros((LANES,), jnp.int32)

    def scan_indices(idx_hbm, list_v):
        pltpu.sync_copy(idx_hbm, idx_v.at[pl.ds(0, BATCH)])

        def body(g, n):
            v = idx_v[pl.ds(g * LANES, LANES)]
            m = (v >= lo) & (v < hi)
            entry = ((v - lo) << 14) | (g * LANES + lane_iota)
            plsc.store_compressed(list_v.at[pl.ds(n, LANES)], entry, mask=m)
            cnt = plsc.all_reduce_population_count(m)
            return n + cnt[0]

        n = lax.fori_loop(0, BATCH // LANES, body, jnp.int32(0))
        list_v[pl.ds(n, LANES)] = zeros_i
        return n

    nu = scan_indices(uidx_hbm, ulist_v)
    ni = scan_indices(iidx_hbm, ilist_v)

    def bucket_sort(src_v, n, dst_v, cnt_v, base_v):
        cnt_v[pl.ds(0, LANES)] = zeros_i
        cnt_v[pl.ds(LANES, LANES)] = zeros_i
        ngroups = (n + LANES - 1) // LANES

        def hist(g, carry):
            e = src_v[pl.ds(g * LANES, LANES)]
            valid = (g * LANES + lane_iota) < n
            buk = lax.shift_right_logical(e, 24)
            plsc.addupdate_scatter(cnt_v, [buk], ones_i, mask=valid)
            return carry

        lax.fori_loop(0, ngroups, hist, 0)

        c0 = cnt_v[pl.ds(0, LANES)]
        c1 = cnt_v[pl.ds(LANES, LANES)]
        incl0 = plsc.cumsum(c0)
        incl1 = plsc.cumsum(c1)
        b0 = incl0 - c0
        b1 = (incl1 - c1) + incl0[LANES - 1]
        base_v[pl.ds(0, LANES)] = b0
        base_v[pl.ds(LANES, LANES)] = b1
        cur_v[pl.ds(0, LANES)] = b0
        cur_v[pl.ds(LANES, LANES)] = b1

        def place(g, carry):
            e = src_v[pl.ds(g * LANES, LANES)]
            buk = lax.shift_right_logical(e, 24)
            for j in range(LANES):
                m_j = (lane_iota == 0) & ((g * LANES + j) < n)
                buk_j = jnp.full((LANES,), buk[j], jnp.int32)
                p = plsc.load_gather(cur_v, [buk_j])
                plsc.store_scatter(
                    dst_v, [jnp.full((LANES,), p[0], jnp.int32)],
                    jnp.full((LANES,), e[j], jnp.int32), mask=m_j)
                plsc.addupdate_scatter(cur_v, [buk_j], ones_i, mask=m_j)
            return carry

        lax.fori_loop(0, ngroups, place, 0)
        dst_v[pl.ds(n, LANES)] = zeros_i

    bucket_sort(ulist_v, nu, idx_v, ucnt_v, ubase_v)
    bucket_sort(ilist_v, ni, ulist_v, icnt_v, ibase_v)
    slist_u = idx_v
    slist_i = ulist_v

    def cond_carry(pred, fn, st):
        return lax.cond(pred, fn, lambda s: s, st)

    def emit_lane(j, m, u_loc, b, buf, s_lo, out_hbm, st):
        def fire(st2):
            def drain(st3):
                pltpu.make_async_copy(
                    stage_v.at[j], out_hbm.at[pl.ds(0, D)], sems.at[j]
                ).wait()
                return st3

            st2 = cond_carry(((st2 >> j) & 1) == 1, drain, st2)
            col = jnp.full((LANES,), u_loc[j] - s_lo, jnp.int32)
            v_lo = plsc.load_gather(buf, [lane_iota, col])
            v_hi = plsc.load_gather(buf, [lane_iota + LANES, col])
            stage_v[j, pl.ds(0, LANES)] = v_lo
            stage_v[j, pl.ds(LANES, LANES)] = v_hi
            pltpu.async_copy(
                stage_v.at[j], out_hbm.at[pl.ds(b[j] * D, D)], sems.at[j])
            return st2 | (1 << j)

        return cond_carry(m[j] == 1, fire, st)

    def scan_bucket(list_v, n, cnt_v, base_v, buk, buf, ncols, s_lo,
                    out_hbm, state):
        bukv = jnp.full((LANES,), buk, jnp.int32)
        seg_lo = plsc.load_gather(base_v, [bukv])[0]
        seg_n = plsc.load_gather(cnt_v, [bukv])[0]
        g_lo = lax.shift_right_logical(seg_lo, 4)
        g_hi = lax.shift_right_logical(seg_lo + seg_n + LANES - 1, 4)

        def group(g, st):
            e = list_v[pl.ds(g * LANES, LANES)]
            pos = g * LANES + lane_iota
            valid = (pos >= seg_lo) & (pos < seg_lo + seg_n)
            u_loc = lax.shift_right_logical(e, 14)
            b = e & 0x3FFF
            m = valid & (u_loc >= s_lo) & (u_loc < s_lo + ncols)
            m_i = m.astype(jnp.int32)

            def emit(st2):
                st3 = st2
                for j in range(LANES):
                    st3 = emit_lane(j, m_i, u_loc, b, buf, s_lo, out_hbm,
                                    st3)
                return st3

            return cond_carry(jnp.any(m), emit, st)

        return lax.fori_loop(g_lo, g_hi, group, state)

    def do_step(c0, s_lo, ncols, buk, ub_dst, ib_dst, st):
        cu = pltpu.async_copy(utab_hbm.at[:, c0], ub_dst, usem)
        ci = pltpu.async_copy(itab_hbm.at[:, c0], ib_dst, isem)
        cu.wait()
        st = scan_bucket(slist_u, nu, ucnt_v, ubase_v, buk, ubuf, ncols,
                         s_lo, uflat_hbm, st)
        ci.wait()
        st = scan_bucket(slist_i, ni, icnt_v, ibase_v, buk, ibuf, ncols,
                         s_lo, iflat_hbm, st)
        return st

    def step_body(s, st):
        return do_step(pl.ds(lo + s * STEP, STEP), s * STEP, STEP, s,
                       ubuf, ibuf, st)

    state = lax.fori_loop(0, FULL_STEPS, step_body, jnp.int32(0))

    def normal_end(st):
        return do_step(pl.ds(lo + FULL_STEPS * STEP, 512),
                       FULL_STEPS * STEP, 512, FULL_STEPS,
                       ubuf.at[:, pl.ds(0, 512)],
                       ibuf.at[:, pl.ds(0, 512)], st)

    def last_end(st):
        st = do_step(pl.ds(lo + FULL_STEPS * STEP, STEP),
                     FULL_STEPS * STEP, STEP, FULL_STEPS, ubuf, ibuf, st)
        s_lo = TAIL_LO - LAST_LO
        pltpu.sync_copy(utab_hbm.at[:, pl.ds(TAIL_LO, TAIL_COLS)], tail_v)
        st = scan_bucket(slist_u, nu, ucnt_v, ubase_v, NBUK - 1, tail_v,
                         TAIL_COLS, s_lo, uflat_hbm, st)
        pltpu.sync_copy(itab_hbm.at[:, pl.ds(TAIL_LO, TAIL_COLS)], tail_v)
        st = scan_bucket(slist_i, ni, icnt_v, ibase_v, NBUK - 1, tail_v,
                         TAIL_COLS, s_lo, iflat_hbm, st)
        return st

    state = lax.cond(last, last_end, normal_end, state)

    def final_drain(st):
        for j in range(LANES):
            def drain(s3, j=j):
                pltpu.make_async_copy(
                    stage_v.at[j], uflat_hbm.at[pl.ds(0, D)], sems.at[j]
                ).wait()
                return s3

            st = cond_carry(((st >> j) & 1) == 1, drain, st)
        return st

    final_drain(state)


def _combine_body(uidx_hbm, iidx_hbm, uflat_hbm, iflat_hbm, fcw_hbm,
                  fcb_hbm, out_hbm, uvals_v, ivals_v, w_v, b_v, out_v, sem):
    wid = lax.axis_index("s") * NC + lax.axis_index("c")
    base = wid * B_PER_W

    cu = pltpu.async_copy(
        uflat_hbm.at[pl.ds(base * D, B_PER_W * D)], uvals_v, sem)
    ci = pltpu.async_copy(
        iflat_hbm.at[pl.ds(base * D, B_PER_W * D)], ivals_v, sem)
    pltpu.sync_copy(fcw_hbm, w_v)
    pltpu.sync_copy(fcb_hbm, b_v)
    cu.wait()
    ci.wait()

    w_lo = w_v[pl.ds(0, LANES)]
    w_hi = w_v[pl.ds(LANES, LANES)]
    bias = b_v[pl.ds(0, LANES)][0]
    row_iota = lax.iota(jnp.int32, LANES) * D

    def group(g, carry):
        acc = jnp.zeros((LANES,), jnp.float32)
        base_flat = g * (LANES * D)
        for d in range(D):
            idxv = row_iota + (base_flat + d)
            gu = plsc.load_gather(uvals_v, [idxv])
            gi = plsc.load_gather(ivals_v, [idxv])
            w_d = w_lo[d] if d < LANES else w_hi[d - LANES]
            acc = acc + gu * gi * w_d
        out_v[pl.ds(g * LANES, LANES)] = acc + bias
        return carry

    lax.fori_loop(0, B_PER_W // LANES, group, 0)

    pltpu.sync_copy(out_v, out_hbm.at[pl.ds(base, B_PER_W)])


def kernel(user_indices, item_indices, user_table, item_table, fc_w, fc_b):
    mesh = plsc.VectorSubcoreMesh(core_axis_name="c", subcore_axis_name="s")
    params = pltpu.CompilerParams(needs_layout_passes=False)

    select = pl.kernel(
        _select_body,
        out_type=(
            jax.ShapeDtypeStruct((BATCH * D,), jnp.float32),
            jax.ShapeDtypeStruct((BATCH * D,), jnp.float32),
        ),
        mesh=mesh,
        compiler_params=params,
        scratch_types=[
            pltpu.VMEM((LIST_CAP,), jnp.int32),
            pltpu.VMEM((LIST_CAP,), jnp.int32),
            pltpu.VMEM((LIST_CAP,), jnp.int32),
            pltpu.VMEM((D, STEP), jnp.float32),
            pltpu.VMEM((D, STEP), jnp.float32),
            pltpu.VMEM((D, TAIL_COLS), jnp.float32),
            pltpu.VMEM((LANES, D), jnp.float32),
            pltpu.VMEM((NBUK,), jnp.int32),
            pltpu.VMEM((NBUK,), jnp.int32),
            pltpu.VMEM((NBUK,), jnp.int32),
            pltpu.VMEM((NBUK,), jnp.int32),
            pltpu.VMEM((NBUK,), jnp.int32),
            pltpu.SemaphoreType.DMA((LANES,)),
            pltpu.SemaphoreType.DMA,
            pltpu.SemaphoreType.DMA,
        ],
    )
    uflat, iflat = select(user_indices, item_indices,
                          user_table.T, item_table.T)

    combine = pl.kernel(
        _combine_body,
        out_type=jax.ShapeDtypeStruct((BATCH,), jnp.float32),
        mesh=mesh,
        compiler_params=params,
        scratch_types=[
            pltpu.VMEM((B_PER_W * D,), jnp.float32),
            pltpu.VMEM((B_PER_W * D,), jnp.float32),
            pltpu.VMEM((D,), jnp.float32),
            pltpu.VMEM((LANES,), jnp.float32),
            pltpu.VMEM((B_PER_W,), jnp.float32),
            pltpu.SemaphoreType.DMA,
        ],
    )
    out = combine(user_indices, item_indices, uflat, iflat,
                  jnp.reshape(fc_w, (D,)),
                  jnp.broadcast_to(jnp.reshape(fc_b, (1,)), (LANES,)))
    return jnp.reshape(out, (BATCH, 1))


# speedup vs baseline: 3.9289x; 1.0988x over previous
import jax
import jax.numpy as jnp
from jax import lax
from jax.experimental import pallas as pl
from jax.experimental.pallas import tpu as pltpu
from jax.experimental.pallas import tpu_sc as plsc

BATCH = 16384
D = 32
NUM_ROWS = 1000000
NC = 2
NS = 16
NW = NC * NS
LANES = 16
COLS_PER_W = 31232
STEP = 512
FULL_STEPS = 61
LAST_LO = 31 * COLS_PER_W
TAIL_LO = 999936
TAIL_COLS = NUM_ROWS - TAIL_LO
NBUK = 64
LIST_CAP = BATCH + LANES
B_PER_W = BATCH // NW


def _select_body(uidx_hbm, iidx_hbm, utab_hbm, itab_hbm,
                 uflat_hbm, iflat_hbm,
                 idx_v, ulist_v, ilist_v, ubuf0, ubuf1, ibuf0, ibuf1,
                 tail_v, stage_v, ucnt_v, ubase_v, icnt_v, ibase_v, cur_v,
                 sems, usem0, usem1, isem0, isem1):
    wid = lax.axis_index("s") * NC + lax.axis_index("c")
    last = wid == NW - 1
    lo = wid * COLS_PER_W
    hi = jnp.where(last, NUM_ROWS, lo + COLS_PER_W)

    lane_iota = lax.iota(jnp.int32, LANES)
    ones_i = jnp.full((LANES,), 1, jnp.int32)
    zeros_i = jnp.zeros((LANES,), jnp.int32)

    def scan_indices(idx_hbm, list_v):
        pltpu.sync_copy(idx_hbm, idx_v.at[pl.ds(0, BATCH)])

        def body(g, n):
            v = idx_v[pl.ds(g * LANES, LANES)]
            m = (v >= lo) & (v < hi)
            entry = ((v - lo) << 14) | (g * LANES + lane_iota)
            plsc.store_compressed(list_v.at[pl.ds(n, LANES)], entry, mask=m)
            cnt = plsc.all_reduce_population_count(m)
            return n + cnt[0]

        n = lax.fori_loop(0, BATCH // LANES, body, jnp.int32(0))
        list_v[pl.ds(n, LANES)] = zeros_i
        return n

    nu = scan_indices(uidx_hbm, ulist_v)
    ni = scan_indices(iidx_hbm, ilist_v)

    def bucket_sort(src_v, n, dst_v, cnt_v, base_v):
        for k in range(NBUK // LANES):
            cnt_v[pl.ds(k * LANES, LANES)] = zeros_i
        ngroups = (n + LANES - 1) // LANES

        def hist(g, carry):
            e = src_v[pl.ds(g * LANES, LANES)]
            valid = (g * LANES + lane_iota) < n
            buk = lax.shift_right_logical(e, 23)
            plsc.addupdate_scatter(cnt_v, [buk], ones_i, mask=valid)
            return carry

        lax.fori_loop(0, ngroups, hist, 0)

        tot = jnp.int32(0)
        for k in range(NBUK // LANES):
            ck = cnt_v[pl.ds(k * LANES, LANES)]
            incl = plsc.cumsum(ck)
            bk = (incl - ck) + tot
            base_v[pl.ds(k * LANES, LANES)] = bk
            cur_v[pl.ds(k * LANES, LANES)] = bk
            tot = tot + incl[LANES - 1]

        def place(g, carry):
            e = src_v[pl.ds(g * LANES, LANES)]
            buk = lax.shift_right_logical(e, 23)
            for j in range(LANES):
                m_j = (lane_iota == 0) & ((g * LANES + j) < n)
                buk_j = jnp.full((LANES,), buk[j], jnp.int32)
                p = plsc.load_gather(cur_v, [buk_j])
                plsc.store_scatter(
                    dst_v, [jnp.full((LANES,), p[0], jnp.int32)],
                    jnp.full((LANES,), e[j], jnp.int32), mask=m_j)
                plsc.addupdate_scatter(cur_v, [buk_j], ones_i, mask=m_j)
            return carry

        lax.fori_loop(0, ngroups, place, 0)
        dst_v[pl.ds(n, LANES)] = zeros_i

    bucket_sort(ulist_v, nu, idx_v, ucnt_v, ubase_v)
    bucket_sort(ilist_v, ni, ulist_v, icnt_v, ibase_v)
    slist_u = idx_v
    slist_i = ulist_v

    def cond_carry(pred, fn, st):
        return lax.cond(pred, fn, lambda s: s, st)

    def emit_lane(j, m, u_loc, b, buf, s_lo, out_hbm, st):
        def fire(st2):
            def drain(st3):
                pltpu.make_async_copy(
                    stage_v.at[j], out_hbm.at[pl.ds(0, D)], sems.at[j]
                ).wait()
                return st3

            st2 = cond_carry(((st2 >> j) & 1) == 1, drain, st2)
            col = jnp.full((LANES,), u_loc[j] - s_lo, jnp.int32)
            v_lo = plsc.load_gather(buf, [lane_iota, col])
            v_hi = plsc.load_gather(buf, [lane_iota + LANES, col])
            stage_v[j, pl.ds(0, LANES)] = v_lo
            stage_v[j, pl.ds(LANES, LANES)] = v_hi
            pltpu.async_copy(
                stage_v.at[j], out_hbm.at[pl.ds(b[j] * D, D)], sems.at[j])
            return st2 | (1 << j)

        return cond_carry(m[j] == 1, fire, st)

    def scan_bucket(list_v, n, cnt_v, base_v, buk, buf, ncols, s_lo,
                    out_hbm, state):
        bukv = jnp.full((LANES,), buk, jnp.int32)
        seg_lo = plsc.load_gather(base_v, [bukv])[0]
        seg_n = plsc.load_gather(cnt_v, [bukv])[0]
        g_lo = lax.shift_right_logical(seg_lo, 4)
        g_hi = lax.shift_right_logical(seg_lo + seg_n + LANES - 1, 4)

        def group(g, st):
            e = list_v[pl.ds(g * LANES, LANES)]
            pos = g * LANES + lane_iota
            valid = (pos >= seg_lo) & (pos < seg_lo + seg_n)
            u_loc = lax.shift_right_logical(e, 14)
            b = e & 0x3FFF
            m = valid & (u_loc >= s_lo) & (u_loc < s_lo + ncols)
            m_i = m.astype(jnp.int32)

            def emit(st2):
                st3 = st2
                for j in range(LANES):
                    st3 = emit_lane(j, m_i, u_loc, b, buf, s_lo, out_hbm,
                                    st3)
                return st3

            return cond_carry(jnp.any(m), emit, st)

        return lax.fori_loop(g_lo, g_hi, group, state)

    nsteps = jnp.where(last, FULL_STEPS + 1, FULL_STEPS)
    ubufs = (ubuf0, ubuf1)
    ibufs = (ibuf0, ibuf1)
    usems = (usem0, usem1)
    isems = (isem0, isem1)

    def fire(s, p):
        c0 = pl.ds(lo + s * STEP, STEP)
        pltpu.async_copy(utab_hbm.at[:, c0], ubufs[p], usems[p])
        pltpu.async_copy(itab_hbm.at[:, c0], ibufs[p], isems[p])

    def drain_scan(s, p, st):
        pltpu.make_async_copy(
            utab_hbm.at[:, pl.ds(0, STEP)], ubufs[p], usems[p]).wait()
        st = scan_bucket(slist_u, nu, ucnt_v, ubase_v, s, ubufs[p], STEP,
                         s * STEP, uflat_hbm, st)
        pltpu.make_async_copy(
            itab_hbm.at[:, pl.ds(0, STEP)], ibufs[p], isems[p]).wait()
        st = scan_bucket(slist_i, ni, icnt_v, ibase_v, s, ibufs[p], STEP,
                         s * STEP, iflat_hbm, st)
        return st

    fire(0, 0)

    def step_pair(sp, st):
        s0 = 2 * sp
        s1 = 2 * sp + 1

        def f1(c):
            fire(s1, 1)
            return c

        lax.cond(s1 < nsteps, f1, lambda c: c, 0)
        st = drain_scan(s0, 0, st)

        def f2(c):
            fire(s0 + 2, 0)
            return c

        lax.cond(s0 + 2 < nsteps, f2, lambda c: c, 0)

        def odd(st2):
            return drain_scan(s1, 1, st2)

        return cond_carry(s1 < nsteps, odd, st)

    state = lax.fori_loop(0, (FULL_STEPS + 2) // 2, step_pair,
                          jnp.int32(0))

    def tail(st):
        s_lo = TAIL_LO - LAST_LO
        pltpu.sync_copy(utab_hbm.at[:, pl.ds(TAIL_LO, TAIL_COLS)], tail_v)
        st = scan_bucket(slist_u, nu, ucnt_v, ubase_v, 62, tail_v,
                         TAIL_COLS, s_lo, uflat_hbm, st)
        pltpu.sync_copy(itab_hbm.at[:, pl.ds(TAIL_LO, TAIL_COLS)], tail_v)
        st = scan_bucket(slist_i, ni, icnt_v, ibase_v, 62, tail_v,
                         TAIL_COLS, s_lo, iflat_hbm, st)
        return st

    state = cond_carry(last, tail, state)

    def final_drain(st):
        for j in range(LANES):
            def drain(s3, j=j):
                pltpu.make_async_copy(
                    stage_v.at[j], uflat_hbm.at[pl.ds(0, D)], sems.at[j]
                ).wait()
                return s3

            st = cond_carry(((st >> j) & 1) == 1, drain, st)
        return st

    final_drain(state)


def _combine_body(uidx_hbm, iidx_hbm, uflat_hbm, iflat_hbm, fcw_hbm,
                  fcb_hbm, out_hbm, uvals_v, ivals_v, w_v, b_v, out_v, sem):
    wid = lax.axis_index("s") * NC + lax.axis_index("c")
    base = wid * B_PER_W

    cu = pltpu.async_copy(
        uflat_hbm.at[pl.ds(base * D, B_PER_W * D)], uvals_v, sem)
    ci = pltpu.async_copy(
        iflat_hbm.at[pl.ds(base * D, B_PER_W * D)], ivals_v, sem)
    pltpu.sync_copy(fcw_hbm, w_v)
    pltpu.sync_copy(fcb_hbm, b_v)
    cu.wait()
    ci.wait()

    w_lo = w_v[pl.ds(0, LANES)]
    w_hi = w_v[pl.ds(LANES, LANES)]
    bias = b_v[pl.ds(0, LANES)][0]
    row_iota = lax.iota(jnp.int32, LANES) * D

    def group(g, carry):
        acc = jnp.zeros((LANES,), jnp.float32)
        base_flat = g * (LANES * D)
        for d in range(D):
            idxv = row_iota + (base_flat + d)
            gu = plsc.load_gather(uvals_v, [idxv])
            gi = plsc.load_gather(ivals_v, [idxv])
            w_d = w_lo[d] if d < LANES else w_hi[d - LANES]
            acc = acc + gu * gi * w_d
        out_v[pl.ds(g * LANES, LANES)] = acc + bias
        return carry

    lax.fori_loop(0, B_PER_W // LANES, group, 0)

    pltpu.sync_copy(out_v, out_hbm.at[pl.ds(base, B_PER_W)])


def kernel(user_indices, item_indices, user_table, item_table, fc_w, fc_b):
    mesh = plsc.VectorSubcoreMesh(core_axis_name="c", subcore_axis_name="s")
    params = pltpu.CompilerParams(needs_layout_passes=False)

    select = pl.kernel(
        _select_body,
        out_type=(
            jax.ShapeDtypeStruct((BATCH * D,), jnp.float32),
            jax.ShapeDtypeStruct((BATCH * D,), jnp.float32),
        ),
        mesh=mesh,
        compiler_params=params,
        scratch_types=[
            pltpu.VMEM((LIST_CAP,), jnp.int32),
            pltpu.VMEM((LIST_CAP,), jnp.int32),
            pltpu.VMEM((LIST_CAP,), jnp.int32),
            pltpu.VMEM((D, STEP), jnp.float32),
            pltpu.VMEM((D, STEP), jnp.float32),
            pltpu.VMEM((D, STEP), jnp.float32),
            pltpu.VMEM((D, STEP), jnp.float32),
            pltpu.VMEM((D, TAIL_COLS), jnp.float32),
            pltpu.VMEM((LANES, D), jnp.float32),
            pltpu.VMEM((NBUK,), jnp.int32),
            pltpu.VMEM((NBUK,), jnp.int32),
            pltpu.VMEM((NBUK,), jnp.int32),
            pltpu.VMEM((NBUK,), jnp.int32),
            pltpu.VMEM((NBUK,), jnp.int32),
            pltpu.SemaphoreType.DMA((LANES,)),
            pltpu.SemaphoreType.DMA,
            pltpu.SemaphoreType.DMA,
            pltpu.SemaphoreType.DMA,
            pltpu.SemaphoreType.DMA,
        ],
    )
    uflat, iflat = select(user_indices, item_indices,
                          user_table.T, item_table.T)

    combine = pl.kernel(
        _combine_body,
        out_type=jax.ShapeDtypeStruct((BATCH,), jnp.float32),
        mesh=mesh,
        compiler_params=params,
        scratch_types=[
            pltpu.VMEM((B_PER_W * D,), jnp.float32),
            pltpu.VMEM((B_PER_W * D,), jnp.float32),
            pltpu.VMEM((D,), jnp.float32),
            pltpu.VMEM((LANES,), jnp.float32),
            pltpu.VMEM((B_PER_W,), jnp.float32),
            pltpu.SemaphoreType.DMA,
        ],
    )
    out = combine(user_indices, item_indices, uflat, iflat,
                  jnp.reshape(fc_w, (D,)),
                  jnp.broadcast_to(jnp.reshape(fc_b, (1,)), (LANES,)))
    return jnp.reshape(out, (BATCH, 1))
